# Initial kernel scaffold; baseline (speedup 1.0000x reference)
import functools

import jax
import jax.numpy as jnp
from jax import lax
from jax.experimental import pallas as pl
from jax.experimental.pallas import tpu as pltpu
from jax.experimental.pallas import tpu_sc as plsc

B = 1024
S = 50
C = 26
D = 128
V = 256
LOW = -3.0
HIGH = 3.0

NC = 2
NS = 16
NW = NC * NS
BPW = B // NW

ROW = S * C
ROWP = 1312
NRV = ROWP // 16
BINS = C * V
MAGIC = 12582912.0


def _sc_hist(xp_flat):
    mesh = plsc.VectorSubcoreMesh(core_axis_name="c", subcore_axis_name="s")

    @functools.partial(
        pl.kernel,
        out_type=jax.ShapeDtypeStruct((B * BINS,), jnp.float32),
        mesh=mesh,
        scratch_types=[
            pltpu.VMEM((BPW * ROWP,), jnp.float32),
            pltpu.VMEM((2 * BINS,), jnp.float32),
            pltpu.SemaphoreType.DMA,
            pltpu.SemaphoreType.DMA,
        ],
    )
    def hist(x_hbm, cnt_hbm, x_v, bins_v, sem0, sem1):
        wid = lax.axis_index("s") * NC + lax.axis_index("c")
        base_b = wid * BPW
        pltpu.sync_copy(x_hbm.at[pl.ds(base_b * ROWP, BPW * ROWP)], x_v)

        lane = lax.iota(jnp.int32, 16)
        ones = jnp.full((16,), 1.0, jnp.float32)
        zeros = jnp.zeros((16,), jnp.float32)
        sems = (sem0, sem1)

        def batch_body(i, carry):
            for par in range(2):
                bb = i * 2 + par
                pbase = par * BINS

                @pl.when(i > 0)
                def _wait():
                    pltpu.make_async_copy(
                        bins_v.at[pl.ds(par * BINS, BINS)],
                        cnt_hbm.at[pl.ds(0, BINS)],
                        sems[par],
                    ).wait()

                def zero_body(z, c2):
                    zo = pbase + z * 128
                    for k in range(8):
                        bins_v[pl.ds(zo + k * 16, 16)] = zeros
                    return c2

                lax.fori_loop(0, BINS // 128, zero_body, 0)

                xoff = bb * ROWP
                for r in range(NRV):
                    v = x_v[pl.ds(xoff + r * 16, 16)]
                    t = ((v - LOW) / (HIGH - LOW)) * float(V - 1)
                    q = (t + MAGIC) - MAGIC
                    q = jnp.minimum(jnp.maximum(q, 0.0), float(V - 1))
                    idx = q.astype(jnp.int32)
                    ch = lax.rem(lane + ((r * 16) % C), C)
                    addr = pbase + ch * V + idx
                    if (r + 1) * 16 <= ROW:
                        plsc.addupdate_scatter(bins_v, [addr], ones)
                    else:
                        m = lane < (ROW - r * 16)
                        plsc.addupdate_scatter(bins_v, [addr], ones, mask=m)

                pltpu.async_copy(
                    bins_v.at[pl.ds(pbase, BINS)],
                    cnt_hbm.at[pl.ds((base_b + bb) * BINS, BINS)],
                    sems[par],
                )
            return carry

        lax.fori_loop(0, BPW // 2, batch_body, 0)

        for par in range(2):
            pltpu.make_async_copy(
                bins_v.at[pl.ds(par * BINS, BINS)],
                cnt_hbm.at[pl.ds(0, BINS)],
                sems[par],
            ).wait()

    return hist(xp_flat)


def _tc_matmul(counts2d, weight):
    M = B * C
    BM = 2048

    def body(c_ref, w_ref, o_ref):
        acc = lax.dot_general(
            c_ref[...], w_ref[...],
            dimension_numbers=(((1,), (0,)), ((), ())),
            preferred_element_type=jnp.float32,
            precision=lax.Precision.HIGHEST,
        )
        o_ref[...] = acc / float(S)

    return pl.pallas_call(
        body,
        grid=(M // BM,),
        in_specs=[
            pl.BlockSpec((BM, V), lambda i: (i, 0)),
            pl.BlockSpec((V, D), lambda i: (0, 0)),
        ],
        out_specs=pl.BlockSpec((BM, D), lambda i: (i, 0)),
        out_shape=jax.ShapeDtypeStruct((M, D), jnp.float32),
    )(counts2d, weight)


def kernel(x, weight):
    xp = jnp.pad(x.reshape(B, ROW), ((0, 0), (0, ROWP - ROW)))
    counts = _sc_hist(xp.reshape(-1))
    out2d = _tc_matmul(counts.reshape(B * C, V), weight)
    return out2d.reshape(B, C, D)


# speedup vs baseline: 26.5658x; 26.5658x over previous
import functools

import jax
import jax.numpy as jnp
from jax import lax
from jax.experimental import pallas as pl
from jax.experimental.pallas import tpu as pltpu
from jax.experimental.pallas import tpu_sc as plsc

B = 1024
S = 50
C = 26
D = 128
V = 256
LOW = -3.0
HIGH = 3.0

NC = 2
NS = 16
NW = NC * NS
BPW = B // NW

ROW = S * C
ROWP = 1312
NRV = ROWP // 16
BINS = C * V
MAGIC = 12582912.0


def _sc_hist(xp_flat):
    mesh = plsc.VectorSubcoreMesh(core_axis_name="c", subcore_axis_name="s")

    @functools.partial(
        pl.kernel,
        out_type=jax.ShapeDtypeStruct((B * BINS,), jnp.float32),
        mesh=mesh,
        scratch_types=[
            pltpu.VMEM((BPW * ROWP,), jnp.float32),
            pltpu.VMEM((2 * BINS,), jnp.float32),
            pltpu.SemaphoreType.DMA,
            pltpu.SemaphoreType.DMA,
        ],
        compiler_params=pltpu.CompilerParams(needs_layout_passes=False),
    )
    def hist(x_hbm, cnt_hbm, x_v, bins_v, sem0, sem1):
        wid = lax.axis_index("s") * NC + lax.axis_index("c")
        base_b = wid * BPW
        pltpu.sync_copy(x_hbm.at[pl.ds(base_b * ROWP, BPW * ROWP)], x_v)

        lane = lax.iota(jnp.int32, 16)
        ones = jnp.full((16,), 1.0, jnp.float32)
        zeros = jnp.zeros((16,), jnp.float32)
        sems = (sem0, sem1)

        def batch_body(i, carry):
            for par in range(2):
                bb = i * 2 + par
                pbase = par * BINS

                @pl.when(i > 0)
                def _wait():
                    pltpu.make_async_copy(
                        bins_v.at[pl.ds(par * BINS, BINS)],
                        cnt_hbm.at[pl.ds(0, BINS)],
                        sems[par],
                    ).wait()

                def zero_body(z, c2):
                    zo = pbase + z * 128
                    for k in range(8):
                        bins_v[pl.ds(zo + k * 16, 16)] = zeros
                    return c2

                lax.fori_loop(0, BINS // 128, zero_body, 0)

                xoff = bb * ROWP
                for r in range(NRV):
                    v = x_v[pl.ds(xoff + r * 16, 16)]
                    t = ((v - LOW) / (HIGH - LOW)) * float(V - 1)
                    q = (t + MAGIC) - MAGIC
                    q = jnp.minimum(jnp.maximum(q, 0.0), float(V - 1))
                    idx = q.astype(jnp.int32)
                    ch = lax.rem(lane + ((r * 16) % C), C)
                    addr = pbase + ch * V + idx
                    if (r + 1) * 16 <= ROW:
                        plsc.addupdate_scatter(bins_v, [addr], ones)
                    else:
                        m = lane < (ROW - r * 16)
                        plsc.addupdate_scatter(bins_v, [addr], ones, mask=m)

                pltpu.async_copy(
                    bins_v.at[pl.ds(pbase, BINS)],
                    cnt_hbm.at[pl.ds((base_b + bb) * BINS, BINS)],
                    sems[par],
                )
            return carry

        lax.fori_loop(0, BPW // 2, batch_body, 0)

        for par in range(2):
            pltpu.make_async_copy(
                bins_v.at[pl.ds(par * BINS, BINS)],
                cnt_hbm.at[pl.ds(0, BINS)],
                sems[par],
            ).wait()

    return hist(xp_flat)


def _tc_matmul(counts2d, weight):
    M = B * C
    BM = 2048

    def body(c_ref, w_ref, o_ref):
        acc = lax.dot_general(
            c_ref[...], w_ref[...],
            dimension_numbers=(((1,), (0,)), ((), ())),
            preferred_element_type=jnp.float32,
            precision=lax.Precision.HIGHEST,
        )
        o_ref[...] = acc / float(S)

    return pl.pallas_call(
        body,
        grid=(M // BM,),
        in_specs=[
            pl.BlockSpec((BM, V), lambda i: (i, 0)),
            pl.BlockSpec((V, D), lambda i: (0, 0)),
        ],
        out_specs=pl.BlockSpec((BM, D), lambda i: (i, 0)),
        out_shape=jax.ShapeDtypeStruct((M, D), jnp.float32),
    )(counts2d, weight)


def kernel(x, weight):
    xp = jnp.pad(x.reshape(B, ROW), ((0, 0), (0, ROWP - ROW)))
    counts = _sc_hist(xp.reshape(-1))
    out2d = _tc_matmul(counts.reshape(B * C, V), weight)
    return out2d.reshape(B, C, D)


# speedup vs baseline: 30.8182x; 1.1601x over previous
import functools

import jax
import jax.numpy as jnp
from jax import lax
from jax.experimental import pallas as pl
from jax.experimental.pallas import tpu as pltpu
from jax.experimental.pallas import tpu_sc as plsc

B = 1024
S = 50
C = 26
D = 128
V = 256
LOW = -3.0
HIGH = 3.0

NC = 2
NS = 16
NW = NC * NS
BPW = B // NW

ROW = S * C
ROWP = 1312
NRV = ROWP // 16
BINS = C * V


def _tc_quant(x2):
    BM = 128

    def body(x_ref, o_ref):
        v = x_ref[...]
        t = ((v - LOW) / (HIGH - LOW)) * float(V - 1)
        q = jnp.clip(jnp.round(t), 0.0, float(V - 1))
        idx = q.astype(jnp.int32)
        ch = lax.rem(lax.broadcasted_iota(jnp.int32, (BM, ROW), 1), C)
        o_ref[...] = jnp.zeros((BM, ROWP), jnp.int32)
        o_ref[:, : ROW] = ch * V + idx

    return pl.pallas_call(
        body,
        grid=(B // BM,),
        in_specs=[pl.BlockSpec((BM, ROW), lambda i: (i, 0))],
        out_specs=pl.BlockSpec((BM, ROWP), lambda i: (i, 0)),
        out_shape=jax.ShapeDtypeStruct((B, ROWP), jnp.int32),
    )(x2)


def _sc_hist(addr_flat):
    mesh = plsc.VectorSubcoreMesh(core_axis_name="c", subcore_axis_name="s")

    @functools.partial(
        pl.kernel,
        out_type=jax.ShapeDtypeStruct((B * BINS,), jnp.float32),
        mesh=mesh,
        scratch_types=[
            pltpu.VMEM((BPW * ROWP,), jnp.int32),
            pltpu.VMEM((2 * BINS,), jnp.float32),
            pltpu.SemaphoreType.DMA,
            pltpu.SemaphoreType.DMA,
        ],
        compiler_params=pltpu.CompilerParams(needs_layout_passes=False),
    )
    def hist(a_hbm, cnt_hbm, a_v, bins_v, sem0, sem1):
        wid = lax.axis_index("s") * NC + lax.axis_index("c")
        base_b = wid * BPW
        pltpu.sync_copy(a_hbm.at[pl.ds(base_b * ROWP, BPW * ROWP)], a_v)

        lane = lax.iota(jnp.int32, 16)
        ones = jnp.full((16,), 1.0, jnp.float32)
        nones = jnp.full((16,), -1.0, jnp.float32)
        zeros = jnp.zeros((16,), jnp.float32)
        sems = (sem0, sem1)
        tail_mask = lane < (ROW - (NRV - 1) * 16)

        def zero_body(z, c2):
            for k in range(8):
                bins_v[pl.ds(z * 128 + k * 16, 16)] = zeros
            return c2

        lax.fori_loop(0, 2 * BINS // 128, zero_body, 0)

        def scat(bb, pbase, val):
            aoff = bb * ROWP
            for r in range(NRV):
                a = a_v[pl.ds(aoff + r * 16, 16)] + pbase
                if (r + 1) * 16 <= ROW:
                    plsc.addupdate_scatter(bins_v, [a], val)
                else:
                    plsc.addupdate_scatter(bins_v, [a], val, mask=tail_mask)

        def batch_body(i, carry):
            for par in range(2):
                bb = i * 2 + par
                pbase = par * BINS

                @pl.when(i > 0)
                def _clear():
                    pltpu.make_async_copy(
                        bins_v.at[pl.ds(par * BINS, BINS)],
                        cnt_hbm.at[pl.ds(0, BINS)],
                        sems[par],
                    ).wait()
                    scat(bb - 2, pbase, nones)

                scat(bb, pbase, ones)

                pltpu.async_copy(
                    bins_v.at[pl.ds(pbase, BINS)],
                    cnt_hbm.at[pl.ds((base_b + bb) * BINS, BINS)],
                    sems[par],
                )
            return carry

        lax.fori_loop(0, BPW // 2, batch_body, 0)

        for par in range(2):
            pltpu.make_async_copy(
                bins_v.at[pl.ds(par * BINS, BINS)],
                cnt_hbm.at[pl.ds(0, BINS)],
                sems[par],
            ).wait()

    return hist(addr_flat)


def _tc_matmul(counts2d, weight):
    M = B * C
    BM = 2048

    def body(c_ref, w_ref, o_ref):
        acc = lax.dot_general(
            c_ref[...], w_ref[...],
            dimension_numbers=(((1,), (0,)), ((), ())),
            preferred_element_type=jnp.float32,
            precision=lax.Precision.HIGHEST,
        )
        o_ref[...] = acc / float(S)

    return pl.pallas_call(
        body,
        grid=(M // BM,),
        in_specs=[
            pl.BlockSpec((BM, V), lambda i: (i, 0)),
            pl.BlockSpec((V, D), lambda i: (0, 0)),
        ],
        out_specs=pl.BlockSpec((BM, D), lambda i: (i, 0)),
        out_shape=jax.ShapeDtypeStruct((M, D), jnp.float32),
    )(counts2d, weight)


def kernel(x, weight):
    addr = _tc_quant(x.reshape(B, ROW))
    counts = _sc_hist(addr.reshape(-1))
    out2d = _tc_matmul(counts.reshape(B * C, V), weight)
    return out2d.reshape(B, C, D)
